# baseline (speedup 1.0000x reference)
import jax
import jax.numpy as jnp
from jax import lax
from jax.experimental import pallas as pl
from jax.experimental.pallas import tpu as pltpu
from jax.experimental.pallas import tpu_sc as plsc

N = 10000
D = 128
M = 64
E = 320000

NCORES = 2
NSUB = 16
NW = NCORES * NSUB
CHUNK = 128
ROWS = NW * 80
EPAD = ROWS * CHUNK
NPAD = 10240
FG = 4
GW = N * FG
AW = NPAD * FG
RB = 1000

_DROWS = ROWS // NW
_AROWS = ROWS // NCORES


def _mesh():
    return plsc.VectorSubcoreMesh(core_axis_name="c", subcore_axis_name="s")


def _zero_ref(ref, nwords, unroll=8):
    z = jnp.zeros((16,), jnp.float32)

    def _z(i, carry):
        for u in range(unroll):
            ref[pl.ds((i * unroll + u) * 16, 16)] = z
        return carry

    lax.fori_loop(0, nwords // (16 * unroll), _z, 0)


_CW = 2048


def _sc_deg_body(dst_hbm, out_hbm, idx0_v, idx1_v, acc_v, sem0, sem1):
    cid = lax.axis_index("c")
    sid = lax.axis_index("s")
    wid = sid * NCORES + cid
    e0 = wid * _DROWS * CHUNK

    _zero_ref(acc_v, NPAD)
    ones = jnp.ones((16,), jnp.float32)
    sems = (sem0, sem1)
    bufs = (idx0_v, idx1_v)

    def _start(c, b):
        pltpu.make_async_copy(
            dst_hbm.at[pl.ds(e0 + c * _CW, _CW)], bufs[b],
            sems[b]).start()

    def _proc(c, b):
        pltpu.make_async_copy(
            dst_hbm.at[pl.ds(e0 + c * _CW, _CW)], bufs[b],
            sems[b]).wait()

        def _vec(i, carry):
            for u in range(8):
                q = i * 8 + u
                t = bufs[b][pl.ds(q * 16, 16)]
                plsc.addupdate_scatter(acc_v, [t], ones)
            return carry

        lax.fori_loop(0, _CW // (16 * 8), _vec, 0)

    _start(0, 0)
    _start(1, 1)
    _proc(0, 0)
    _start(2, 0)
    _proc(1, 1)
    _start(3, 1)
    _proc(2, 0)
    _start(4, 0)
    _proc(3, 1)
    _proc(4, 0)
    pltpu.sync_copy(acc_v, out_hbm.at[wid])


def _sc_deg(dst2d):
    return pl.kernel(
        _sc_deg_body,
        out_type=jax.ShapeDtypeStruct((NW, NPAD), jnp.float32),
        mesh=_mesh(),
        compiler_params=pltpu.CompilerParams(needs_layout_passes=False),
        scratch_types=[
            pltpu.VMEM((_CW,), jnp.int32),
            pltpu.VMEM((_CW,), jnp.int32),
            pltpu.VMEM((NPAD,), jnp.float32),
            pltpu.SemaphoreType.DMA,
            pltpu.SemaphoreType.DMA,
        ],
    )(dst2d)


def _sc_agg_body(g_hbm, src_hbm, dst_hbm, out_hbm,
                 gsub_v, acc_v, sidx0_v, sidx1_v, didx0_v, didx1_v,
                 ss0, ss1, sd0, sd1):
    cid = lax.axis_index("c")
    sid = lax.axis_index("s")
    e0 = cid * _AROWS * CHUNK

    _zero_ref(acc_v, AW)
    pltpu.sync_copy(g_hbm.at[sid], gsub_v)
    sss = (ss0, ss1)
    sds = (sd0, sd1)
    sbufs = (sidx0_v, sidx1_v)
    dbufs = (didx0_v, didx1_v)

    def _start(c, b):
        pltpu.make_async_copy(
            src_hbm.at[pl.ds(e0 + c * _CW, _CW)], sbufs[b],
            sss[b]).start()
        pltpu.make_async_copy(
            dst_hbm.at[pl.ds(e0 + c * _CW, _CW)], dbufs[b],
            sds[b]).start()

    def _proc(c, b):
        pltpu.make_async_copy(
            src_hbm.at[pl.ds(e0 + c * _CW, _CW)], sbufs[b],
            sss[b]).wait()
        pltpu.make_async_copy(
            dst_hbm.at[pl.ds(e0 + c * _CW, _CW)], dbufs[b],
            sds[b]).wait()

        def _vec(i, carry):
            for u in range(4):
                q = i * 4 + u
                s = sbufs[b][pl.ds(q * 16, 16)] * FG
                t = dbufs[b][pl.ds(q * 16, 16)] * FG
                for j in range(FG):
                    vals = plsc.load_gather(gsub_v, [s + j])
                    plsc.addupdate_scatter(acc_v, [t + j], vals)
            return carry

        lax.fori_loop(0, _CW // (16 * 4), _vec, 0)

    _start(0, 0)
    _start(1, 1)

    def _body(jo, carry):
        for b in range(2):
            c = jo * 2 + b
            _proc(c, b)
            _start(c + 2, b)
        return carry

    lax.fori_loop(0, 39, _body, 0)
    _proc(78, 0)
    _proc(79, 1)
    pltpu.sync_copy(acc_v, out_hbm.at[cid, sid])


def _sc_agg(ggrp, src2d, dst2d):
    return pl.kernel(
        _sc_agg_body,
        out_type=jax.ShapeDtypeStruct((NCORES, NSUB, AW), jnp.float32),
        mesh=_mesh(),
        compiler_params=pltpu.CompilerParams(needs_layout_passes=False),
        scratch_types=[
            pltpu.VMEM((GW,), jnp.float32),
            pltpu.VMEM((AW,), jnp.float32),
            pltpu.VMEM((_CW,), jnp.int32),
            pltpu.VMEM((_CW,), jnp.int32),
            pltpu.VMEM((_CW,), jnp.int32),
            pltpu.VMEM((_CW,), jnp.int32),
            pltpu.SemaphoreType.DMA,
            pltpu.SemaphoreType.DMA,
            pltpu.SemaphoreType.DMA,
            pltpu.SemaphoreType.DMA,
        ],
    )(ggrp, src2d, dst2d)


def _tc_b_body(degp_ref, x_ref, w1_ref, g1_ref, dinv_ref):
    d = jnp.sum(degp_ref[...], axis=0) + 1.0
    dv = lax.rsqrt(d)
    h = jnp.dot(x_ref[...], w1_ref[...], preferred_element_type=jnp.float32)
    g1_ref[...] = dv * h
    dinv_ref[...] = dv


def _tc_b(degp, x, W1):
    return pl.pallas_call(
        _tc_b_body,
        grid=(N // RB,),
        in_specs=[
            pl.BlockSpec((NW, RB, 1), lambda i: (0, i, 0)),
            pl.BlockSpec((RB, D), lambda i: (i, 0)),
            pl.BlockSpec((D, M), lambda i: (0, 0)),
        ],
        out_specs=[
            pl.BlockSpec((RB, M), lambda i: (i, 0)),
            pl.BlockSpec((RB, 1), lambda i: (i, 0)),
        ],
        out_shape=[
            jax.ShapeDtypeStruct((N, M), jnp.float32),
            jax.ShapeDtypeStruct((N, 1), jnp.float32),
        ],
    )(degp, x, W1)


def _tc_d_body(sp_ref, g1_ref, dinv_ref, b1_ref, g2_ref):
    t = sp_ref[0] + sp_ref[1] + g1_ref[...]
    dv = dinv_ref[...]
    h = jnp.maximum(dv * t + b1_ref[...], 0.0)
    g2_ref[...] = dv * h


def _tc_d(sp, g1, dinv, b1row):
    return pl.pallas_call(
        _tc_d_body,
        grid=(N // RB,),
        in_specs=[
            pl.BlockSpec((NCORES, RB, M), lambda i: (0, i, 0)),
            pl.BlockSpec((RB, M), lambda i: (i, 0)),
            pl.BlockSpec((RB, 1), lambda i: (i, 0)),
            pl.BlockSpec((1, M), lambda i: (0, 0)),
        ],
        out_specs=pl.BlockSpec((RB, M), lambda i: (i, 0)),
        out_shape=jax.ShapeDtypeStruct((N, M), jnp.float32),
    )(sp, g1, dinv, b1row)


def _tc_f_body(sp_ref, g2_ref, dinv_ref, w2_ref, b2_ref, o_ref):
    t = dinv_ref[...] * (sp_ref[0] + sp_ref[1] + g2_ref[...])
    o_ref[...] = jnp.dot(t, w2_ref[...],
                         preferred_element_type=jnp.float32) + b2_ref[...]


def _tc_f(sp, g2, dinv, W2, b2row):
    return pl.pallas_call(
        _tc_f_body,
        grid=(N // RB,),
        in_specs=[
            pl.BlockSpec((NCORES, RB, M), lambda i: (0, i, 0)),
            pl.BlockSpec((RB, M), lambda i: (i, 0)),
            pl.BlockSpec((RB, 1), lambda i: (i, 0)),
            pl.BlockSpec((M, D), lambda i: (0, 0)),
            pl.BlockSpec((1, D), lambda i: (0, 0)),
        ],
        out_specs=pl.BlockSpec((RB, D), lambda i: (i, 0)),
        out_shape=jax.ShapeDtypeStruct((N, D), jnp.float32),
    )(sp, g2, dinv, W2, b2row)


def _group(g):
    return g.reshape(N, NSUB, FG).transpose(1, 0, 2).reshape(NSUB, GW)


def _ungroup(p):
    return (p.reshape(NCORES, NSUB, NPAD, FG)
            .transpose(0, 2, 1, 3).reshape(NCORES, NPAD, M)[:, :N])


def kernel(x, edge_index, W1, b1, W2, b2):
    src = edge_index[0]
    dst = edge_index[1]
    pad = EPAD - E
    src1d = jnp.concatenate([src, jnp.zeros((pad,), jnp.int32)])
    dst1d = jnp.concatenate([dst, jnp.full((pad,), N, jnp.int32)])

    degp = _sc_deg(dst1d)[:, :N].reshape(NW, N, 1)
    g1, dinv = _tc_b(degp, x, W1)
    s1 = _ungroup(_sc_agg(_group(g1), src1d, dst1d))
    g2 = _tc_d(s1, g1, dinv, b1.reshape(1, M))
    s2 = _ungroup(_sc_agg(_group(g2), src1d, dst1d))
    return _tc_f(s2, g2, dinv, W2, b2.reshape(1, D))


# speedup vs baseline: 1.4213x; 1.4213x over previous
import jax
import jax.numpy as jnp
from jax import lax
from jax.experimental import pallas as pl
from jax.experimental.pallas import tpu as pltpu
from jax.experimental.pallas import tpu_sc as plsc

N = 10000
D = 128
M = 64
E = 320000

NCORES = 2
NSUB = 16
NW = NCORES * NSUB
CHUNK = 128
ROWS = NW * 80
EPAD = ROWS * CHUNK
NPAD = 10240
FG = 4
GW = N * FG
AW = NPAD * FG
RB = 1000

_DROWS = ROWS // NW
_CW = 2048
HW = 32
CA = 104
SROWS = NSUB * CA
TROWS = (ROWS - SROWS) // NCORES
TCHUNKS = TROWS * CHUNK // _CW
SHARE = NPAD // NSUB


def _mesh():
    return plsc.VectorSubcoreMesh(core_axis_name="c", subcore_axis_name="s")


def _zero_ref(ref, nwords, unroll=8):
    z = jnp.zeros((16,), jnp.float32)

    def _z(i, carry):
        for u in range(unroll):
            ref[pl.ds((i * unroll + u) * 16, 16)] = z
        return carry

    lax.fori_loop(0, nwords // (16 * unroll), _z, 0)


_DCW = 2048


def _sc_deg_body(dst_hbm, out_hbm, idx0_v, idx1_v, acc_v, sem0, sem1):
    cid = lax.axis_index("c")
    sid = lax.axis_index("s")
    wid = sid * NCORES + cid
    e0 = wid * _DROWS * CHUNK

    _zero_ref(acc_v, NPAD)
    ones = jnp.ones((16,), jnp.float32)
    sems = (sem0, sem1)
    bufs = (idx0_v, idx1_v)

    def _start(c, b):
        pltpu.make_async_copy(
            dst_hbm.at[pl.ds(e0 + c * _DCW, _DCW)], bufs[b],
            sems[b]).start()

    def _proc(c, b):
        pltpu.make_async_copy(
            dst_hbm.at[pl.ds(e0 + c * _DCW, _DCW)], bufs[b],
            sems[b]).wait()

        def _vec(i, carry):
            for u in range(8):
                q = i * 8 + u
                t = bufs[b][pl.ds(q * 16, 16)]
                plsc.addupdate_scatter(acc_v, [t], ones)
            return carry

        lax.fori_loop(0, _DCW // (16 * 8), _vec, 0)

    _start(0, 0)
    _start(1, 1)
    _proc(0, 0)
    _start(2, 0)
    _proc(1, 1)
    _start(3, 1)
    _proc(2, 0)
    _start(4, 0)
    _proc(3, 1)
    _proc(4, 0)
    pltpu.sync_copy(acc_v, out_hbm.at[wid])


def _sc_deg(dst2d):
    return pl.kernel(
        _sc_deg_body,
        out_type=jax.ShapeDtypeStruct((NW, NPAD), jnp.float32),
        mesh=_mesh(),
        compiler_params=pltpu.CompilerParams(needs_layout_passes=False),
        scratch_types=[
            pltpu.VMEM((_DCW,), jnp.int32),
            pltpu.VMEM((_DCW,), jnp.int32),
            pltpu.VMEM((NPAD,), jnp.float32),
            pltpu.SemaphoreType.DMA,
            pltpu.SemaphoreType.DMA,
        ],
    )(dst2d)


def _sc_agg_body(ghalf_hbm, ggrp_hbm, epk_hbm, src1_hbm, dst1_hbm,
                 osp_hbm, otl_hbm,
                 gsub_v, acc_v, rows_v, sdq_v,
                 tsidx0_v, tsidx1_v, tdidx0_v, tdidx1_v,
                 gs0, gs1, gs2, gs3, sc0, sc1, sc2, sc3,
                 is0, st0, st1, sd0, sd1,
                 acc_sp):
    cid = lax.axis_index("c")
    sid = lax.axis_index("s")
    te0 = (SROWS + cid * TROWS) * CHUNK

    gsems = (gs0, gs1, gs2, gs3)
    ssems = (sc0, sc1, sc2, sc3)
    tss = (st0, st1)
    tds = (sd0, sd1)
    tsb = (tsidx0_v, tsidx1_v)
    tdb = (tdidx0_v, tdidx1_v)

    _zero_ref(acc_v, AW)
    for r in range(CHUNK):
        for k in range(HW // 16):
            rows_v[0, r, pl.ds(k * 16, 16)] = jnp.zeros((16,), jnp.float32)
    for k in range(SHARE // CHUNK):
        pltpu.sync_copy(rows_v.at[0],
                        acc_sp.at[pl.ds(sid * SHARE + k * CHUNK, CHUNK), :])
    pltpu.sync_copy(ggrp_hbm.at[sid], gsub_v)

    def _tec_start(c, b2):
        pltpu.make_async_copy(
            src1_hbm.at[pl.ds(te0 + c * _CW, _CW)], tsb[b2], tss[b2]).start()
        pltpu.make_async_copy(
            dst1_hbm.at[pl.ds(te0 + c * _CW, _CW)], tdb[b2], tds[b2]).start()

    def _tec_wait(c, b2):
        pltpu.make_async_copy(
            src1_hbm.at[pl.ds(te0 + c * _CW, _CW)], tsb[b2], tss[b2]).wait()
        pltpu.make_async_copy(
            dst1_hbm.at[pl.ds(te0 + c * _CW, _CW)], tdb[b2], tds[b2]).wait()

    def _idx_start(k):
        if k < CA:
            pltpu.make_async_copy(epk_hbm.at[sid * CA + k],
                                  sdq_v.at[k % 4], is0).start()

    def _idx_wait(k):
        if k < CA:
            pltpu.make_async_copy(epk_hbm.at[sid * CA + k],
                                  sdq_v.at[k % 4], is0).wait()

    def _gather_start(k):
        if k < CA:
            pltpu.make_async_copy(ghalf_hbm.at[cid].at[sdq_v.at[k % 4, 0]],
                                  rows_v.at[k % 4], gsems[k % 4]).start()

    def _step(k):
        if k < CA:
            pltpu.make_async_copy(ghalf_hbm.at[cid].at[sdq_v.at[k % 4, 0]],
                                  rows_v.at[k % 4], gsems[k % 4]).wait()
            pltpu.async_copy(rows_v.at[k % 4], acc_sp.at[sdq_v.at[k % 4, 1]],
                             ssems[k % 4], add=True)
        if 0 <= k - 2 < CA:
            pltpu.make_async_copy(rows_v.at[(k - 2) % 4],
                                  acc_sp.at[sdq_v.at[(k - 2) % 4, 1]],
                                  ssems[(k - 2) % 4]).wait()
        _idx_wait(k + 2)
        _gather_start(k + 2)
        _idx_start(k + 3)

    def _tec_vecs(b2, i0, i1):
        def _vec(i, carry):
            for u in range(4):
                q = i * 4 + u
                sv = tsb[b2][pl.ds(q * 16, 16)] * FG
                tv = tdb[b2][pl.ds(q * 16, 16)] * FG
                for j in range(FG):
                    vals = plsc.load_gather(gsub_v, [sv + j])
                    plsc.addupdate_scatter(acc_v, [tv + j], vals)
            return carry

        lax.fori_loop(i0, i1, _vec, 0)

    _tec_start(0, 0)
    _tec_start(1, 1)
    for k in range(3):
        _idx_start(k)
    plsc.subcore_barrier()
    for k in range(2):
        _idx_wait(k)
        _gather_start(k)

    nv = _CW // (16 * 4)
    for ts in range(TCHUNKS):
        b2 = ts % 2
        _tec_wait(ts, b2)
        steps = list(range(CA * ts // TCHUNKS, CA * (ts + 1) // TCHUNKS))
        cuts = [nv * (i + 1) // len(steps) for i in range(len(steps))]
        lo = 0
        for k, hi in zip(steps, cuts):
            _step(k)
            _tec_vecs(b2, lo, hi)
            lo = hi
        if ts + 2 < TCHUNKS:
            _tec_start(ts + 2, b2)
    _step(CA)
    _step(CA + 1)

    plsc.subcore_barrier()
    pltpu.sync_copy(acc_sp.at[pl.ds(sid * SHARE, SHARE), :],
                    osp_hbm.at[cid, pl.ds(sid * SHARE, SHARE), :])
    pltpu.sync_copy(acc_v, otl_hbm.at[cid, sid])


def _sc_agg(ghalf, ggrp, epk, src1d, dst1d):
    return pl.kernel(
        _sc_agg_body,
        out_type=[
            jax.ShapeDtypeStruct((NCORES, NPAD, HW), jnp.float32),
            jax.ShapeDtypeStruct((NCORES, NSUB, AW), jnp.float32),
        ],
        mesh=_mesh(),
        compiler_params=pltpu.CompilerParams(
            needs_layout_passes=False, use_tc_tiling_on_sc=False),
        scratch_types=[
            pltpu.VMEM((GW,), jnp.float32),
            pltpu.VMEM((AW,), jnp.float32),
            pltpu.VMEM((4, CHUNK, HW), jnp.float32),
            pltpu.VMEM((4, 2, CHUNK), jnp.int32),
            pltpu.VMEM((_CW,), jnp.int32),
            pltpu.VMEM((_CW,), jnp.int32),
            pltpu.VMEM((_CW,), jnp.int32),
            pltpu.VMEM((_CW,), jnp.int32),
        ] + [pltpu.SemaphoreType.DMA] * 13 + [
            pltpu.VMEM_SHARED((NPAD, HW), jnp.float32),
        ],
    )(ghalf, ggrp, epk, src1d, dst1d)


def _tc_b_body(degp_ref, x_ref, w1_ref, g1_ref, dinv_ref):
    d = jnp.sum(degp_ref[...], axis=0) + 1.0
    dv = lax.rsqrt(d)
    h = jnp.dot(x_ref[...], w1_ref[...], preferred_element_type=jnp.float32)
    g1_ref[...] = dv * h
    dinv_ref[...] = dv


def _tc_b(degp, x, W1):
    return pl.pallas_call(
        _tc_b_body,
        grid=(N // RB,),
        in_specs=[
            pl.BlockSpec((NW, RB, 1), lambda i: (0, i, 0)),
            pl.BlockSpec((RB, D), lambda i: (i, 0)),
            pl.BlockSpec((D, M), lambda i: (0, 0)),
        ],
        out_specs=[
            pl.BlockSpec((RB, M), lambda i: (i, 0)),
            pl.BlockSpec((RB, 1), lambda i: (i, 0)),
        ],
        out_shape=[
            jax.ShapeDtypeStruct((N, M), jnp.float32),
            jax.ShapeDtypeStruct((N, 1), jnp.float32),
        ],
    )(degp, x, W1)


def _tc_d_body(sp_ref, tp_ref, g1_ref, dinv_ref, b1_ref, g2_ref):
    t = sp_ref[...] + tp_ref[0] + tp_ref[1] + g1_ref[...]
    dv = dinv_ref[...]
    h = jnp.maximum(dv * t + b1_ref[...], 0.0)
    g2_ref[...] = dv * h


def _tc_d(sp, tp, g1, dinv, b1row):
    return pl.pallas_call(
        _tc_d_body,
        grid=(N // RB,),
        in_specs=[
            pl.BlockSpec((RB, M), lambda i: (i, 0)),
            pl.BlockSpec((NCORES, RB, M), lambda i: (0, i, 0)),
            pl.BlockSpec((RB, M), lambda i: (i, 0)),
            pl.BlockSpec((RB, 1), lambda i: (i, 0)),
            pl.BlockSpec((1, M), lambda i: (0, 0)),
        ],
        out_specs=pl.BlockSpec((RB, M), lambda i: (i, 0)),
        out_shape=jax.ShapeDtypeStruct((N, M), jnp.float32),
    )(sp, tp, g1, dinv, b1row)


def _tc_f_body(sp_ref, tp_ref, g2_ref, dinv_ref, w2_ref, b2_ref, o_ref):
    t = dinv_ref[...] * (sp_ref[...] + tp_ref[0] + tp_ref[1] + g2_ref[...])
    o_ref[...] = jnp.dot(t, w2_ref[...],
                         preferred_element_type=jnp.float32) + b2_ref[...]


def _tc_f(sp, tp, g2, dinv, W2, b2row):
    return pl.pallas_call(
        _tc_f_body,
        grid=(N // RB,),
        in_specs=[
            pl.BlockSpec((RB, M), lambda i: (i, 0)),
            pl.BlockSpec((NCORES, RB, M), lambda i: (0, i, 0)),
            pl.BlockSpec((RB, M), lambda i: (i, 0)),
            pl.BlockSpec((RB, 1), lambda i: (i, 0)),
            pl.BlockSpec((M, D), lambda i: (0, 0)),
            pl.BlockSpec((1, D), lambda i: (0, 0)),
        ],
        out_specs=pl.BlockSpec((RB, D), lambda i: (i, 0)),
        out_shape=jax.ShapeDtypeStruct((N, D), jnp.float32),
    )(sp, tp, g2, dinv, W2, b2row)


def _half(g):
    return g.reshape(N, NCORES, HW).transpose(1, 0, 2)


def _unhalf(sp):
    return sp.transpose(1, 0, 2).reshape(NPAD, M)[:N]


def _group(g):
    return g.reshape(N, NSUB, FG).transpose(1, 0, 2).reshape(NSUB, GW)


def _ungroup(p):
    return (p.reshape(NCORES, NSUB, NPAD, FG)
            .transpose(0, 2, 1, 3).reshape(NCORES, NPAD, M)[:, :N])


def kernel(x, edge_index, W1, b1, W2, b2):
    src = edge_index[0]
    dst = edge_index[1]
    pad = EPAD - E
    src1d = jnp.concatenate([src, jnp.zeros((pad,), jnp.int32)])
    dst1d = jnp.concatenate([dst, jnp.full((pad,), N, jnp.int32)])

    degp = _sc_deg(dst1d)[:, :N].reshape(NW, N, 1)
    g1, dinv = _tc_b(degp, x, W1)
    epk = jnp.stack([src1d.reshape(ROWS, CHUNK),
                     dst1d.reshape(ROWS, CHUNK)], axis=1)
    sp1, tp1 = _sc_agg(_half(g1), _group(g1), epk, src1d, dst1d)
    g2 = _tc_d(_unhalf(sp1), _ungroup(tp1), g1, dinv, b1.reshape(1, M))
    sp2, tp2 = _sc_agg(_half(g2), _group(g2), epk, src1d, dst1d)
    return _tc_f(_unhalf(sp2), _ungroup(tp2), g2, dinv, W2, b2.reshape(1, D))
